# baseline (speedup 1.0000x reference)
import functools

import jax
import jax.numpy as jnp
from jax.experimental import pallas as pl
from jax.experimental.pallas import tpu as pltpu

N = 10000
T = 8
H = 128
R2 = 1.0


def _gru_body(x_ref, wih_ref, whh_ref, bih_ref, bhh_ref, out_ref, hlast_ref):
    bn = x_ref.shape[0]
    wih = wih_ref[...]
    whh = whh_ref[...]
    bih = bih_ref[...]
    bhh = bhh_ref[...]
    h = jnp.zeros((bn, H), dtype=jnp.float32)
    for t in range(T):
        x_t = x_ref[:, t, :]
        gi = jax.lax.dot_general(x_t, wih, (((1,), (1,)), ((), ())),
                                 preferred_element_type=jnp.float32) + bih
        gh = jax.lax.dot_general(h, whh, (((1,), (1,)), ((), ())),
                                 preferred_element_type=jnp.float32) + bhh
        r = jax.nn.sigmoid(gi[:, 0:H] + gh[:, 0:H])
        z = jax.nn.sigmoid(gi[:, H:2 * H] + gh[:, H:2 * H])
        n = jnp.tanh(gi[:, 2 * H:3 * H] + r * gh[:, 2 * H:3 * H])
        h = (1.0 - z) * n + z * h
        out_ref[:, t, :] = h
    hlast_ref[...] = h


def _gru_call(x_seq, W_ih, W_hh, b_ih, b_hh, block_n, interpret=False):
    n = x_seq.shape[0]
    grid = (n // block_n,)
    return pl.pallas_call(
        _gru_body,
        grid=grid,
        in_specs=[
            pl.BlockSpec((block_n, T, x_seq.shape[2]), lambda i: (i, 0, 0)),
            pl.BlockSpec(W_ih.shape, lambda i: (0, 0)),
            pl.BlockSpec(W_hh.shape, lambda i: (0, 0)),
            pl.BlockSpec((1, 3 * H), lambda i: (0, 0)),
            pl.BlockSpec((1, 3 * H), lambda i: (0, 0)),
        ],
        out_specs=[
            pl.BlockSpec((block_n, T, H), lambda i: (i, 0, 0)),
            pl.BlockSpec((block_n, H), lambda i: (i, 0)),
        ],
        out_shape=[
            jax.ShapeDtypeStruct((n, T, H), jnp.float32),
            jax.ShapeDtypeStruct((n, H), jnp.float32),
        ],
        interpret=interpret,
    )(x_seq, W_ih, W_hh, b_ih.reshape(1, -1), b_hh.reshape(1, -1))


def _agg_body(valid_ref, posj_ref, posti_ref, hi_ref, hj_ref, wf_ref, bf_ref,
              wp_ref, bp_ref, fused_ref, pred_ref, acc_ref, cnt_ref, *,
              num_i, bi, bj):
    i = pl.program_id(1)
    j = pl.program_id(0)

    @pl.when(i == 0)
    def _init():
        acc_ref[...] = jnp.zeros_like(acc_ref)
        cnt_ref[...] = jnp.zeros_like(cnt_ref)

    @pl.when(valid_ref[j, i] == 1)
    def _accumulate():
        posj = posj_ref[...]
        posti = posti_ref[...]
        sqj = jnp.sum(posj * posj, axis=1, keepdims=True)
        sqi_row = jnp.sum(posti * posti, axis=0, keepdims=True)
        cross = jax.lax.dot_general(posj.astype(jnp.bfloat16),
                                    posti.astype(jnp.bfloat16),
                                    (((1,), (0,)), ((), ())),
                                    preferred_element_type=jnp.float32)
        d2 = sqj + sqi_row - 2.0 * cross
        gj = j * bj + jax.lax.broadcasted_iota(jnp.int32, (bj, bi), 0)
        gi = i * bi + jax.lax.broadcasted_iota(jnp.int32, (bj, bi), 1)
        maskf = jnp.where((d2 <= R2) & (gj != gi), 1.0, 0.0)
        acc_ref[...] += jax.lax.dot_general(maskf.astype(jnp.bfloat16),
                                            hi_ref[...],
                                            (((1,), (0,)), ((), ())),
                                            preferred_element_type=jnp.float32)
        cnt_ref[...] += jnp.sum(maskf, axis=1, keepdims=True)

    @pl.when(i == num_i - 1)
    def _epilogue():
        msg = acc_ref[...] / jnp.maximum(cnt_ref[...], 1.0)
        hj = hj_ref[...]
        wf = wf_ref[...]
        f1 = jax.lax.dot_general(hj, wf[:, 0:H], (((1,), (1,)), ((), ())),
                                 preferred_element_type=jnp.float32)
        f2 = jax.lax.dot_general(msg, wf[:, H:2 * H], (((1,), (1,)), ((), ())),
                                 preferred_element_type=jnp.float32)
        fused = jnp.maximum(f1 + f2 + bf_ref[...], 0.0)
        fused_ref[...] = fused
        pred_ref[...] = jax.lax.dot_general(fused, wp_ref[...],
                                            (((1,), (1,)), ((), ())),
                                            preferred_element_type=jnp.float32
                                            ) + bp_ref[...]


def _agg_call(pos, h, valid, W_fuse, b_fuse, W_pred, b_pred, block_j,
              block_i, interpret=False):
    n = pos.shape[0]
    nj, ni = n // block_j, n // block_i
    body = functools.partial(_agg_body, num_i=ni, bi=block_i, bj=block_j)
    return pl.pallas_call(
        body,
        grid=(nj, ni),
        in_specs=[
            pl.BlockSpec(memory_space=pltpu.SMEM),
            pl.BlockSpec((block_j, 2), lambda j, i: (j, 0)),
            pl.BlockSpec((2, block_i), lambda j, i: (0, i)),
            pl.BlockSpec((block_i, H), lambda j, i: (i, 0)),
            pl.BlockSpec((block_j, H), lambda j, i: (j, 0)),
            pl.BlockSpec(W_fuse.shape, lambda j, i: (0, 0)),
            pl.BlockSpec((1, H), lambda j, i: (0, 0)),
            pl.BlockSpec(W_pred.shape, lambda j, i: (0, 0)),
            pl.BlockSpec((1, 2), lambda j, i: (0, 0)),
        ],
        out_specs=[
            pl.BlockSpec((block_j, H), lambda j, i: (j, 0)),
            pl.BlockSpec((block_j, 2), lambda j, i: (j, 0)),
        ],
        out_shape=[
            jax.ShapeDtypeStruct((n, H), jnp.float32),
            jax.ShapeDtypeStruct((n, 2), jnp.float32),
        ],
        scratch_shapes=[
            pltpu.VMEM((block_j, H), jnp.float32),
            pltpu.VMEM((block_j, 1), jnp.float32),
        ],
        interpret=interpret,
    )(valid, pos, pos.T, h.astype(jnp.bfloat16), h, W_fuse,
      b_fuse.reshape(1, -1), W_pred, b_pred.reshape(1, -1))


def _build_schedule(pos, h_last, block):
    n = pos.shape[0]
    np_ = ((n + block - 1) // block) * block
    nb = np_ // block
    cx = jnp.clip(jnp.floor(pos[:, 0]), 0, 31).astype(jnp.int32)
    cy = jnp.clip(jnp.floor(pos[:, 1]), 0, 31).astype(jnp.int32)
    key = jnp.concatenate([cy * 32 + cx,
                           jnp.full((np_ - n,), 4096, jnp.int32)])
    sidx = jnp.argsort(key)
    pos_p = jnp.pad(pos, ((0, np_ - n), (0, 0)), constant_values=62.6)
    h_p = jnp.pad(h_last, ((0, np_ - n), (0, 0)))
    pos_s = pos_p[sidx]
    h_s = h_p[sidx]
    pb = pos_s.reshape(nb, block, 2)
    bmin = pb.min(axis=1)
    bmax = pb.max(axis=1)
    gap = jnp.maximum(
        jnp.maximum(bmin[:, None, :] - bmax[None, :, :],
                    bmin[None, :, :] - bmax[:, None, :]), 0.0)
    gap2 = jnp.sum(gap * gap, axis=-1)
    crossmax = (bmax[:, None, 0] * bmax[None, :, 0]
                + bmax[:, None, 1] * bmax[None, :, 1])
    bound = R2 + crossmax * (2.0 ** -7) * 1.01 + 1e-3
    valid = (gap2 <= bound).astype(jnp.int32)
    inv = jnp.zeros((np_,), jnp.int32).at[sidx].set(
        jnp.arange(np_, dtype=jnp.int32))
    return pos_s, h_s, valid, inv


def kernel(x_seq, pos_seq, W_ih, W_hh, b_ih, b_hh, W_fuse, b_fuse, W_pred,
           b_pred):
    temporal_out, h_last = _gru_call(x_seq, W_ih, W_hh, b_ih, b_hh, 1000)
    pos = pos_seq[:, -1, :]
    n = pos.shape[0]
    block = 512
    pos_s, h_s, valid, inv = _build_schedule(pos, h_last, block)
    fused_s, preds_s = _agg_call(pos_s, h_s, valid, W_fuse, b_fuse, W_pred,
                                 b_pred, block, block)
    return (preds_s[inv[:n]], temporal_out, fused_s[inv[:n]])


# speedup vs baseline: 1.3548x; 1.3548x over previous
import functools

import jax
import jax.numpy as jnp
from jax.experimental import pallas as pl
from jax.experimental.pallas import tpu as pltpu

N = 10000
T = 8
H = 128
R2 = 1.0


def _gru_body(x_ref, wih_ref, whh_ref, bih_ref, bhh_ref, out_ref, hlast_ref):
    bn = x_ref.shape[0]
    wih = wih_ref[...]
    whh = whh_ref[...]
    bih = bih_ref[...]
    bhh = bhh_ref[...]
    h = jnp.zeros((bn, H), dtype=jnp.float32)
    for t in range(T):
        x_t = x_ref[:, t, :]
        gi = jax.lax.dot_general(x_t, wih, (((1,), (1,)), ((), ())),
                                 preferred_element_type=jnp.float32) + bih
        gh = jax.lax.dot_general(h, whh, (((1,), (1,)), ((), ())),
                                 preferred_element_type=jnp.float32) + bhh
        r = jax.nn.sigmoid(gi[:, 0:H] + gh[:, 0:H])
        z = jax.nn.sigmoid(gi[:, H:2 * H] + gh[:, H:2 * H])
        n = jnp.tanh(gi[:, 2 * H:3 * H] + r * gh[:, 2 * H:3 * H])
        h = (1.0 - z) * n + z * h
        out_ref[:, t, :] = h
    hlast_ref[...] = h


def _gru_call(x_seq, W_ih, W_hh, b_ih, b_hh, block_n, interpret=False):
    n = x_seq.shape[0]
    grid = (n // block_n,)
    return pl.pallas_call(
        _gru_body,
        grid=grid,
        in_specs=[
            pl.BlockSpec((block_n, T, x_seq.shape[2]), lambda i: (i, 0, 0)),
            pl.BlockSpec(W_ih.shape, lambda i: (0, 0)),
            pl.BlockSpec(W_hh.shape, lambda i: (0, 0)),
            pl.BlockSpec((1, 3 * H), lambda i: (0, 0)),
            pl.BlockSpec((1, 3 * H), lambda i: (0, 0)),
        ],
        out_specs=[
            pl.BlockSpec((block_n, T, H), lambda i: (i, 0, 0)),
            pl.BlockSpec((block_n, H), lambda i: (i, 0)),
        ],
        out_shape=[
            jax.ShapeDtypeStruct((n, T, H), jnp.float32),
            jax.ShapeDtypeStruct((n, H), jnp.float32),
        ],
        interpret=interpret,
    )(x_seq, W_ih, W_hh, b_ih.reshape(1, -1), b_hh.reshape(1, -1))


def _agg_body(valid_ref, posj_ref, posti_ref, hi_ref, hj_ref, wf_ref, bf_ref,
              wp_ref, bp_ref, fused_ref, pred_ref, acc_ref, cnt_ref, *,
              num_i, bi, bj):
    i = pl.program_id(1)
    j = pl.program_id(0)

    @pl.when(i == 0)
    def _init():
        acc_ref[...] = jnp.zeros_like(acc_ref)
        cnt_ref[...] = jnp.zeros_like(cnt_ref)

    @pl.when(valid_ref[j, i] == 1)
    def _accumulate():
        posj = posj_ref[...]
        posti = posti_ref[...]
        sqj = jnp.sum(posj * posj, axis=1, keepdims=True)
        sqi_row = jnp.sum(posti * posti, axis=0, keepdims=True)
        cross = jax.lax.dot_general(posj.astype(jnp.bfloat16),
                                    posti.astype(jnp.bfloat16),
                                    (((1,), (0,)), ((), ())),
                                    preferred_element_type=jnp.float32)
        d2 = sqj + sqi_row - 2.0 * cross
        gj = j * bj + jax.lax.broadcasted_iota(jnp.int32, (bj, bi), 0)
        gi = i * bi + jax.lax.broadcasted_iota(jnp.int32, (bj, bi), 1)
        maskf = jnp.where((d2 <= R2) & (gj != gi), 1.0, 0.0)
        acc_ref[...] += jax.lax.dot_general(maskf.astype(jnp.bfloat16),
                                            hi_ref[...],
                                            (((1,), (0,)), ((), ())),
                                            preferred_element_type=jnp.float32)
        cnt_ref[...] += jnp.sum(maskf, axis=1, keepdims=True)

    @pl.when(i == num_i - 1)
    def _epilogue():
        msg = acc_ref[...] / jnp.maximum(cnt_ref[...], 1.0)
        hj = hj_ref[...]
        wf = wf_ref[...]
        f1 = jax.lax.dot_general(hj, wf[:, 0:H], (((1,), (1,)), ((), ())),
                                 preferred_element_type=jnp.float32)
        f2 = jax.lax.dot_general(msg, wf[:, H:2 * H], (((1,), (1,)), ((), ())),
                                 preferred_element_type=jnp.float32)
        fused = jnp.maximum(f1 + f2 + bf_ref[...], 0.0)
        fused_ref[...] = fused
        pred_ref[...] = jax.lax.dot_general(fused, wp_ref[...],
                                            (((1,), (1,)), ((), ())),
                                            preferred_element_type=jnp.float32
                                            ) + bp_ref[...]


def _agg_call(pos, h, valid, W_fuse, b_fuse, W_pred, b_pred, block_j,
              block_i, interpret=False):
    n = pos.shape[0]
    nj, ni = n // block_j, n // block_i
    body = functools.partial(_agg_body, num_i=ni, bi=block_i, bj=block_j)
    return pl.pallas_call(
        body,
        grid=(nj, ni),
        in_specs=[
            pl.BlockSpec(memory_space=pltpu.SMEM),
            pl.BlockSpec((block_j, 2), lambda j, i: (j, 0)),
            pl.BlockSpec((2, block_i), lambda j, i: (0, i)),
            pl.BlockSpec((block_i, H), lambda j, i: (i, 0)),
            pl.BlockSpec((block_j, H), lambda j, i: (j, 0)),
            pl.BlockSpec(W_fuse.shape, lambda j, i: (0, 0)),
            pl.BlockSpec((1, H), lambda j, i: (0, 0)),
            pl.BlockSpec(W_pred.shape, lambda j, i: (0, 0)),
            pl.BlockSpec((1, 2), lambda j, i: (0, 0)),
        ],
        out_specs=[
            pl.BlockSpec((block_j, H), lambda j, i: (j, 0)),
            pl.BlockSpec((block_j, 2), lambda j, i: (j, 0)),
        ],
        out_shape=[
            jax.ShapeDtypeStruct((n, H), jnp.float32),
            jax.ShapeDtypeStruct((n, 2), jnp.float32),
        ],
        scratch_shapes=[
            pltpu.VMEM((block_j, H), jnp.float32),
            pltpu.VMEM((block_j, 1), jnp.float32),
        ],
        interpret=interpret,
    )(valid, pos, pos.T, h.astype(jnp.bfloat16), h, W_fuse,
      b_fuse.reshape(1, -1), W_pred, b_pred.reshape(1, -1))


def _build_schedule(pos, h_last, block):
    n = pos.shape[0]
    np_ = ((n + block - 1) // block) * block
    nb = np_ // block
    cx = jnp.clip(jnp.floor(pos[:, 0]), 0, 31).astype(jnp.int32)
    cy = jnp.clip(jnp.floor(pos[:, 1]), 0, 31).astype(jnp.int32)
    key = jnp.concatenate([cy * 32 + cx,
                           jnp.full((np_ - n,), 4096, jnp.int32)])
    sidx = jnp.argsort(key)
    pos_p = jnp.pad(pos, ((0, np_ - n), (0, 0)), constant_values=62.6)
    h_p = jnp.pad(h_last, ((0, np_ - n), (0, 0)))
    pos_s = pos_p[sidx]
    h_s = h_p[sidx]
    pb = pos_s.reshape(nb, block, 2)
    bmin = pb.min(axis=1)
    bmax = pb.max(axis=1)
    gap = jnp.maximum(
        jnp.maximum(bmin[:, None, :] - bmax[None, :, :],
                    bmin[None, :, :] - bmax[:, None, :]), 0.0)
    gap2 = jnp.sum(gap * gap, axis=-1)
    crossmax = (bmax[:, None, 0] * bmax[None, :, 0]
                + bmax[:, None, 1] * bmax[None, :, 1])
    bound = R2 + crossmax * (2.0 ** -7) * 1.01 + 1e-3
    valid = (gap2 <= bound).astype(jnp.int32)
    inv = jnp.zeros((np_,), jnp.int32).at[sidx].set(
        jnp.arange(np_, dtype=jnp.int32))
    return pos_s, h_s, valid, inv


def kernel(x_seq, pos_seq, W_ih, W_hh, b_ih, b_hh, W_fuse, b_fuse, W_pred,
           b_pred):
    temporal_out, h_last = _gru_call(x_seq, W_ih, W_hh, b_ih, b_hh, 1000)
    pos = pos_seq[:, -1, :]
    n = pos.shape[0]
    block = 1024
    pos_s, h_s, valid, inv = _build_schedule(pos, h_last, block)
    fused_s, preds_s = _agg_call(pos_s, h_s, valid, W_fuse, b_fuse, W_pred,
                                 b_pred, block, block)
    return (preds_s[inv[:n]], temporal_out, fused_s[inv[:n]])


# speedup vs baseline: 1.4377x; 1.0612x over previous
import functools

import jax
import jax.numpy as jnp
from jax.experimental import pallas as pl
from jax.experimental.pallas import tpu as pltpu

N = 10000
T = 8
H = 128
R2 = 1.0


def _gru_body(x_ref, wih_ref, whh_ref, bih_ref, bhh_ref, out_ref, hlast_ref):
    bn = x_ref.shape[0]
    wih = wih_ref[...]
    whh = whh_ref[...]
    bih = bih_ref[...]
    bhh = bhh_ref[...]
    h = jnp.zeros((bn, H), dtype=jnp.float32)
    for t in range(T):
        x_t = x_ref[:, t, :]
        gi = jax.lax.dot_general(x_t, wih, (((1,), (1,)), ((), ())),
                                 preferred_element_type=jnp.float32) + bih
        gh = jax.lax.dot_general(h, whh, (((1,), (1,)), ((), ())),
                                 preferred_element_type=jnp.float32) + bhh
        r = jax.nn.sigmoid(gi[:, 0:H] + gh[:, 0:H])
        z = jax.nn.sigmoid(gi[:, H:2 * H] + gh[:, H:2 * H])
        n = jnp.tanh(gi[:, 2 * H:3 * H] + r * gh[:, 2 * H:3 * H])
        h = (1.0 - z) * n + z * h
        out_ref[:, t, :] = h
    hlast_ref[...] = h


def _gru_call(x_seq, W_ih, W_hh, b_ih, b_hh, block_n, interpret=False):
    n = x_seq.shape[0]
    grid = (n // block_n,)
    return pl.pallas_call(
        _gru_body,
        grid=grid,
        in_specs=[
            pl.BlockSpec((block_n, T, x_seq.shape[2]), lambda i: (i, 0, 0)),
            pl.BlockSpec(W_ih.shape, lambda i: (0, 0)),
            pl.BlockSpec(W_hh.shape, lambda i: (0, 0)),
            pl.BlockSpec((1, 3 * H), lambda i: (0, 0)),
            pl.BlockSpec((1, 3 * H), lambda i: (0, 0)),
        ],
        out_specs=[
            pl.BlockSpec((block_n, T, H), lambda i: (i, 0, 0)),
            pl.BlockSpec((block_n, H), lambda i: (i, 0)),
        ],
        out_shape=[
            jax.ShapeDtypeStruct((n, T, H), jnp.float32),
            jax.ShapeDtypeStruct((n, H), jnp.float32),
        ],
        interpret=interpret,
    )(x_seq, W_ih, W_hh, b_ih.reshape(1, -1), b_hh.reshape(1, -1))


def _agg_body(imap_ref, nvalid_ref, posj_ref, posti_ref, hi_ref, hj_ref,
              wf_ref, bf_ref, wp_ref, bp_ref, fused_ref, pred_ref, acc_ref,
              cnt_ref, *, num_i, bi, bj):
    k = pl.program_id(1)
    j = pl.program_id(0)

    @pl.when(k == 0)
    def _init():
        acc_ref[...] = jnp.zeros_like(acc_ref)
        cnt_ref[...] = jnp.zeros_like(cnt_ref)

    @pl.when(k < nvalid_ref[j])
    def _accumulate():
        i = imap_ref[j, k]
        posj = posj_ref[...]
        posti = posti_ref[...]
        sqj = jnp.sum(posj * posj, axis=1, keepdims=True)
        sqi_row = jnp.sum(posti * posti, axis=0, keepdims=True)
        cross = jax.lax.dot_general(posj.astype(jnp.bfloat16),
                                    posti.astype(jnp.bfloat16),
                                    (((1,), (0,)), ((), ())),
                                    preferred_element_type=jnp.float32)
        d2 = sqj + sqi_row - 2.0 * cross
        gj = j * bj + jax.lax.broadcasted_iota(jnp.int32, (bj, bi), 0)
        gi = i * bi + jax.lax.broadcasted_iota(jnp.int32, (bj, bi), 1)
        maskf = jnp.where((d2 <= R2) & (gj != gi), 1.0, 0.0)
        acc_ref[...] += jax.lax.dot_general(maskf.astype(jnp.bfloat16),
                                            hi_ref[...],
                                            (((1,), (0,)), ((), ())),
                                            preferred_element_type=jnp.float32)
        cnt_ref[...] += jnp.sum(maskf, axis=1, keepdims=True)

    @pl.when(k == num_i - 1)
    def _epilogue():
        msg = acc_ref[...] / jnp.maximum(cnt_ref[...], 1.0)
        hj = hj_ref[...]
        wf = wf_ref[...]
        f1 = jax.lax.dot_general(hj, wf[:, 0:H], (((1,), (1,)), ((), ())),
                                 preferred_element_type=jnp.float32)
        f2 = jax.lax.dot_general(msg, wf[:, H:2 * H], (((1,), (1,)), ((), ())),
                                 preferred_element_type=jnp.float32)
        fused = jnp.maximum(f1 + f2 + bf_ref[...], 0.0)
        fused_ref[...] = fused
        pred_ref[...] = jax.lax.dot_general(fused, wp_ref[...],
                                            (((1,), (1,)), ((), ())),
                                            preferred_element_type=jnp.float32
                                            ) + bp_ref[...]


def _agg_call(pos, h, imap, nvalid, W_fuse, b_fuse, W_pred, b_pred, block_j,
              block_i, interpret=False):
    n = pos.shape[0]
    nj, ni = n // block_j, n // block_i
    body = functools.partial(_agg_body, num_i=ni, bi=block_i, bj=block_j)
    grid_spec = pltpu.PrefetchScalarGridSpec(
        num_scalar_prefetch=2,
        grid=(nj, ni),
        in_specs=[
            pl.BlockSpec((block_j, 2), lambda j, k, m, nv: (j, 0)),
            pl.BlockSpec((2, block_i), lambda j, k, m, nv: (0, m[j, k])),
            pl.BlockSpec((block_i, H), lambda j, k, m, nv: (m[j, k], 0)),
            pl.BlockSpec((block_j, H), lambda j, k, m, nv: (j, 0)),
            pl.BlockSpec(W_fuse.shape, lambda j, k, m, nv: (0, 0)),
            pl.BlockSpec((1, H), lambda j, k, m, nv: (0, 0)),
            pl.BlockSpec(W_pred.shape, lambda j, k, m, nv: (0, 0)),
            pl.BlockSpec((1, 2), lambda j, k, m, nv: (0, 0)),
        ],
        out_specs=[
            pl.BlockSpec((block_j, H), lambda j, k, m, nv: (j, 0)),
            pl.BlockSpec((block_j, 2), lambda j, k, m, nv: (j, 0)),
        ],
        scratch_shapes=[
            pltpu.VMEM((block_j, H), jnp.float32),
            pltpu.VMEM((block_j, 1), jnp.float32),
        ],
    )
    return pl.pallas_call(
        body,
        grid_spec=grid_spec,
        out_shape=[
            jax.ShapeDtypeStruct((n, H), jnp.float32),
            jax.ShapeDtypeStruct((n, 2), jnp.float32),
        ],
        interpret=interpret,
    )(imap, nvalid, pos, pos.T, h.astype(jnp.bfloat16), h, W_fuse,
      b_fuse.reshape(1, -1), W_pred, b_pred.reshape(1, -1))


def _build_schedule(pos, h_last, block):
    n = pos.shape[0]
    np_ = ((n + block - 1) // block) * block
    nb = np_ // block
    cx = jnp.clip(jnp.floor(pos[:, 0]), 0, 31).astype(jnp.int32)
    cy = jnp.clip(jnp.floor(pos[:, 1]), 0, 31).astype(jnp.int32)
    key = jnp.concatenate([cy * 32 + cx,
                           jnp.full((np_ - n,), 4096, jnp.int32)])
    sidx = jnp.argsort(key)
    pos_p = jnp.pad(pos, ((0, np_ - n), (0, 0)), constant_values=62.6)
    h_p = jnp.pad(h_last, ((0, np_ - n), (0, 0)))
    pos_s = pos_p[sidx]
    h_s = h_p[sidx]
    pb = pos_s.reshape(nb, block, 2)
    bmin = pb.min(axis=1)
    bmax = pb.max(axis=1)
    gap = jnp.maximum(
        jnp.maximum(bmin[:, None, :] - bmax[None, :, :],
                    bmin[None, :, :] - bmax[:, None, :]), 0.0)
    gap2 = jnp.sum(gap * gap, axis=-1)
    crossmax = (bmax[:, None, 0] * bmax[None, :, 0]
                + bmax[:, None, 1] * bmax[None, :, 1])
    bound = R2 + crossmax * (2.0 ** -7) * 1.01 + 1e-3
    valid = (gap2 <= bound).astype(jnp.int32)
    order = jnp.argsort(-valid, axis=1, stable=True).astype(jnp.int32)
    nvalid = jnp.sum(valid, axis=1, dtype=jnp.int32)
    last = jnp.take_along_axis(order, (nvalid - 1)[:, None], axis=1)
    imap = jnp.where(jnp.arange(nb, dtype=jnp.int32)[None, :]
                     < nvalid[:, None], order, last)
    inv = jnp.zeros((np_,), jnp.int32).at[sidx].set(
        jnp.arange(np_, dtype=jnp.int32))
    return pos_s, h_s, imap, nvalid, inv


def kernel(x_seq, pos_seq, W_ih, W_hh, b_ih, b_hh, W_fuse, b_fuse, W_pred,
           b_pred):
    temporal_out, h_last = _gru_call(x_seq, W_ih, W_hh, b_ih, b_hh, 1000)
    pos = pos_seq[:, -1, :]
    n = pos.shape[0]
    block = 1024
    pos_s, h_s, imap, nvalid, inv = _build_schedule(pos, h_last, block)
    fused_s, preds_s = _agg_call(pos_s, h_s, imap, nvalid, W_fuse, b_fuse,
                                 W_pred, b_pred, block, block)
    return (preds_s[inv[:n]], temporal_out, fused_s[inv[:n]])
